# Initial kernel scaffold; baseline (speedup 1.0000x reference)
import jax
import jax.numpy as jnp
from jax.experimental import pallas as pl


def kernel(s_feats, t_feats, logits_S, logits_T, labels, W1, gamma, beta, W2, seg_queue, pix_queue, seg_ptr, pix_ptr):
    raise NotImplementedError("write your pallas kernel here")


# speedup vs baseline: 9.8997x; 9.8997x over previous
import functools

import jax
import jax.numpy as jnp
import numpy as np
from jax import lax
from jax.experimental import pallas as pl
from jax.experimental.pallas import tpu as pltpu
from jax.experimental.pallas import tpu_sc as plsc

NUM_CLASSES = 19
IGNORE = 255
DIM = 256
REGION_MEM = 2000
PIXEL_MEM = 20000
PIXEL_CONTRAST = 4096 // NUM_CLASSES + 1
REGION_CONTRAST = 1024 // NUM_CLASSES + 1
TAU_C = 0.1
KD_T = 1.0
MAX_SAMPLES = 1024
PIX_UPD = 10
LW_PIX = 0.1
LW_REG = 0.1

M = 8192
CHUNK = 512
NCHUNK = M // CHUNK
KPAD = 192
NPIX = NUM_CLASSES * PIXEL_CONTRAST
NREG = NUM_CLASSES * REGION_CONTRAST
NPIX_PAD = 4224
NREG_PAD = 1152
NW = 32
NPIX_SC = 4352
NREG_SC = 1280
PB = NPIX_SC // NW
RB = NREG_SC // NW

_PIDX = np.array([
    19851, 12832, 2748, 10523, 1960, 5101, 10204, 14383, 8490, 8589, 7203,
    13428, 2994, 7745, 16530, 9747, 15513, 10494, 11667, 1697, 16122, 17138,
    15651, 19828, 8375, 10461, 6872, 18476, 9449, 10646, 8416, 797, 11263,
    2182, 9573, 10059, 15041, 6983, 3116, 18154, 3046, 12007, 8180, 13800,
    14128, 3207, 18959, 12575, 5344, 12351, 15909, 2261, 13268, 13183, 18122,
    2529, 4684, 10331, 11933, 4549, 8970, 8549, 13137, 15150, 15675, 13074,
    19287, 3038, 4685, 14202, 32, 15331, 13996, 19724, 8289, 14748, 3146,
    11400, 8388, 12080, 16497, 886, 5079, 5271, 1386, 6805, 18926, 6182,
    18284, 14273, 17271, 4667, 13937, 17759, 10745, 8206, 1692, 11015, 3746,
    13444, 2580, 2734, 4544, 5468, 12671, 4416, 16991, 11227, 19270, 5295,
    11974, 6850, 9245, 6058, 16590, 14973, 5521, 3692, 3623, 4204, 4224,
    17054, 4744, 15849, 8733, 10963, 2489, 14426, 4747, 17117, 11126, 17410,
    15315, 7495, 3616, 8960, 9836, 1280, 1597, 2322, 15244, 2129, 6593,
    16353, 18690, 8726, 6863, 6085, 17385, 10050, 14322, 10388, 206, 3778,
    11961, 4109, 10799, 9723, 19031, 9039, 19086, 14720, 11385, 12325, 1564,
    1471, 7612, 4989, 4659, 19561, 1843, 9986, 15303, 16629, 6853, 15096,
    15294, 4438, 19374, 1226, 11689, 9025, 16624, 4897, 14948, 13578, 14308,
    17701, 9489, 543, 3926, 9700, 16286, 7649, 19236, 13304, 6473, 13249,
    10943, 6016, 14963, 408, 19324, 16118, 15221, 483, 4915, 12933, 16443,
    2306, 16188, 4682, 18063, 16821, 7018, 5746], dtype=np.int32)

_RIDX = np.array([
    1858, 1255, 1078, 297, 1329, 1302, 1072, 900, 1014, 185, 1354, 1985,
    1053, 678, 1348, 454, 1309, 1361, 1668, 664, 1450, 1031, 15, 318, 859,
    1525, 1146, 89, 253, 606, 1318, 115, 1898, 686, 839, 258, 586, 1826,
    1079, 1474, 1911, 1857, 437, 1831, 1803, 1912, 452, 713, 1083, 892, 1086,
    879, 1446, 1147], dtype=np.int32)


def _dg(a, b, ca, cb):
    return lax.dot_general(a, b, (((ca,), (cb,)), ((), ())),
                           preferred_element_type=jnp.float32)


def _prep_body(lab_ref, s_ref, t_ref, w1_ref, g_ref, b_ref, w2_ref,
               sa_ref, ta_ref, mf_ref, upd_ref, cnt_ref,
               xa, tas, ssum, ssq, segs, updacc, basec, cntcol):
    i = pl.program_id(0)

    @pl.when(i == 0)
    def _init():
        ssum[...] = jnp.zeros_like(ssum)
        ssq[...] = jnp.zeros_like(ssq)
        segs[...] = jnp.zeros_like(segs)
        updacc[...] = jnp.zeros_like(updacc)
        basec[...] = jnp.zeros_like(basec)
        cntcol[...] = jnp.zeros_like(cntcol)

    lab = lab_ref[...]
    tb = t_ref[...]
    tb = tb / (jnp.sqrt(jnp.sum(tb * tb, axis=1, keepdims=True)) + 1e-12)
    cls = lax.broadcasted_iota(jnp.int32, (CHUNK, NUM_CLASSES), 1)
    oh = jnp.where((lab == cls) & (lab != IGNORE), 1.0, 0.0)

    cnt_b = jnp.sum(oh, axis=0, keepdims=True)
    ri = lax.broadcasted_iota(jnp.int32, (CHUNK, CHUNK), 0)
    ci = lax.broadcasted_iota(jnp.int32, (CHUNK, CHUNK), 1)
    tril = jnp.where(ci < ri, 1.0, 0.0)
    excl = _dg(tril, oh, 1, 0) + basec[...]
    basec[...] = basec[...] + cnt_b
    segs[...] = segs[...] + _dg(oh, tb, 0, 0)
    cntcol[...] = cntcol[...] + _dg(oh, jnp.ones((CHUNK, 1), jnp.float32), 0, 0)

    ecls = lax.broadcasted_iota(jnp.int32, (NUM_CLASSES, KPAD), 0)
    ecol = lax.broadcasted_iota(jnp.int32, (NUM_CLASSES, KPAD), 1)
    emat = jnp.where(ecol // PIX_UPD == ecls, 1.0, 0.0)
    clsw = _dg(oh, emat, 1, 0)
    exw = _dg(excl, emat, 1, 0)
    kvec = (lax.broadcasted_iota(jnp.int32, (1, KPAD), 1) % PIX_UPD
            ).astype(jnp.float32)
    sel = clsw * jnp.where(exw == kvec, 1.0, 0.0)
    updacc[...] = updacc[...] + _dg(sel, tb, 0, 0)

    x1 = _dg(s_ref[...], w1_ref[...], 1, 1)
    ssum[...] = ssum[...] + jnp.sum(x1, axis=0, keepdims=True)
    ssq[...] = ssq[...] + jnp.sum(x1 * x1, axis=0, keepdims=True)

    @pl.when(i < MAX_SAMPLES // CHUNK)
    def _store():
        xa[pl.ds(i * CHUNK, CHUNK), :] = x1
        tas[pl.ds(i * CHUNK, CHUNK), :] = tb

    @pl.when(i == NCHUNK - 1)
    def _final():
        mean = ssum[...] / float(M)
        var = ssq[...] / float(M) - mean * mean
        xn = (xa[...] - mean) / jnp.sqrt(var + 1e-5) * g_ref[...] + b_ref[...]
        xn = jnp.maximum(xn, 0.0)
        s2 = _dg(xn, w2_ref[...], 1, 1)
        sa_ref[...] = s2 / (jnp.sqrt(jnp.sum(s2 * s2, axis=1, keepdims=True))
                            + 1e-12)
        ta_ref[...] = tas[...]
        mf = segs[...] / jnp.maximum(cntcol[...], 1.0)
        mf_ref[...] = mf / (jnp.sqrt(jnp.sum(mf * mf, axis=1, keepdims=True))
                            + 1e-12)
        u = updacc[...]
        upd_ref[...] = u / (jnp.sqrt(jnp.sum(u * u, axis=1, keepdims=True))
                            + 1e-12)
        cnt_ref[...] = cntcol[...]


def _prep(lab2d, s_p_in, t_p_in, W1, gamma, beta, W2):
    f32 = jnp.float32
    return pl.pallas_call(
        _prep_body,
        grid=(NCHUNK,),
        in_specs=[
            pl.BlockSpec((CHUNK, 1), lambda i: (i, 0)),
            pl.BlockSpec((CHUNK, 512), lambda i: (i, 0)),
            pl.BlockSpec((CHUNK, DIM), lambda i: (i, 0)),
            pl.BlockSpec((DIM, 512), lambda i: (0, 0)),
            pl.BlockSpec((1, DIM), lambda i: (0, 0)),
            pl.BlockSpec((1, DIM), lambda i: (0, 0)),
            pl.BlockSpec((DIM, DIM), lambda i: (0, 0)),
        ],
        out_specs=[
            pl.BlockSpec((MAX_SAMPLES, DIM), lambda i: (0, 0)),
            pl.BlockSpec((MAX_SAMPLES, DIM), lambda i: (0, 0)),
            pl.BlockSpec((NUM_CLASSES, DIM), lambda i: (0, 0)),
            pl.BlockSpec((KPAD, DIM), lambda i: (0, 0)),
            pl.BlockSpec((NUM_CLASSES, 1), lambda i: (0, 0)),
        ],
        out_shape=[
            jax.ShapeDtypeStruct((MAX_SAMPLES, DIM), f32),
            jax.ShapeDtypeStruct((MAX_SAMPLES, DIM), f32),
            jax.ShapeDtypeStruct((NUM_CLASSES, DIM), f32),
            jax.ShapeDtypeStruct((KPAD, DIM), f32),
            jax.ShapeDtypeStruct((NUM_CLASSES, 1), f32),
        ],
        scratch_shapes=[
            pltpu.VMEM((MAX_SAMPLES, DIM), f32),
            pltpu.VMEM((MAX_SAMPLES, DIM), f32),
            pltpu.VMEM((1, DIM), f32),
            pltpu.VMEM((1, DIM), f32),
            pltpu.VMEM((NUM_CLASSES, DIM), f32),
            pltpu.VMEM((KPAD, DIM), f32),
            pltpu.VMEM((1, NUM_CLASSES), f32),
            pltpu.VMEM((NUM_CLASSES, 1), f32),
        ],
    )(lab2d, s_p_in, t_p_in, W1, gamma, beta, W2)


def _sc_gather(ptab, pidx_pad, rtab, ridx_pad):
    mesh = plsc.VectorSubcoreMesh(core_axis_name="c", subcore_axis_name="s")

    @functools.partial(
        pl.kernel, mesh=mesh,
        out_type=[jax.ShapeDtypeStruct((NPIX_SC, DIM), jnp.float32),
                  jax.ShapeDtypeStruct((NREG_SC, DIM), jnp.float32)],
        scratch_types=[
            pltpu.VMEM((PB,), jnp.int32),
            pltpu.VMEM((PB, DIM), jnp.float32),
            pltpu.VMEM((RB,), jnp.int32),
            pltpu.VMEM((RB, DIM), jnp.float32),
            pltpu.SemaphoreType.DMA,
            pltpu.SemaphoreType.DMA,
        ],
    )
    def gk(ptab_h, pidx_h, rtab_h, ridx_h, outp_h, outr_h,
           pidx_v, prow_v, ridx_v, rrow_v, sem1, sem2):
        wid = lax.axis_index("s") * 2 + lax.axis_index("c")
        pb = wid * PB
        rb = wid * RB
        pltpu.sync_copy(pidx_h.at[pl.ds(pb, PB)], pidx_v)
        cp1 = pltpu.async_copy(ptab_h.at[pidx_v], prow_v, sem1)
        pltpu.sync_copy(ridx_h.at[pl.ds(rb, RB)], ridx_v)
        cp2 = pltpu.async_copy(rtab_h.at[ridx_v], rrow_v, sem2)
        cp1.wait()
        cp2.wait()
        pltpu.sync_copy(prow_v, outp_h.at[pl.ds(pb, PB)])
        pltpu.sync_copy(rrow_v, outr_h.at[pl.ds(rb, RB)])

    return gk(ptab, pidx_pad, rtab, ridx_pad)


def _loss_body(sa_ref, ta_ref, xpb_ref, rp_ref, mp_ref, xrb_ref, rr_ref,
               mr_ref, wa_ref, lp_ref, lr_ref, xps, xrs, accs):
    i = pl.program_id(0)

    @pl.when(i == 0)
    def _init():
        mp = mp_ref[...]
        xps[...] = xpb_ref[...] + mp * (rp_ref[...] - xpb_ref[...])
        mr = mr_ref[...]
        xrs[...] = xrb_ref[...] + mr * (rr_ref[...] - xrb_ref[...])
        accs[0] = 0.0
        accs[1] = 0.0
        accs[2] = 0.0

    sa = sa_ref[...]
    ta = ta_ref[...]
    wa = wa_ref[...]

    def kd_part(x, nvalid, ncols):
        zs = _dg(sa, x, 1, 1) * (1.0 / TAU_C / KD_T)
        zt = _dg(ta, x, 1, 1) * (1.0 / TAU_C / KD_T)
        msk = lax.broadcasted_iota(jnp.int32, (1, ncols), 1) < nvalid
        zs = jnp.where(msk, zs, -1e30)
        zt = jnp.where(msk, zt, -1e30)
        zs = zs - jnp.max(zs, axis=1, keepdims=True)
        zt = zt - jnp.max(zt, axis=1, keepdims=True)
        logps = zs - jnp.log(jnp.sum(jnp.exp(zs), axis=1, keepdims=True))
        logpt = zt - jnp.log(jnp.sum(jnp.exp(zt), axis=1, keepdims=True))
        pt = jnp.exp(logpt)
        kl = jnp.sum(pt * (logpt - logps), axis=1, keepdims=True)
        return jnp.sum(kl * wa)

    vp = kd_part(xps[...], NPIX, NPIX_PAD)
    vr = kd_part(xrs[...], NREG, NREG_PAD)
    accs[0] = accs[0] + vp
    accs[1] = accs[1] + vr
    accs[2] = accs[2] + jnp.sum(wa)

    @pl.when(i == pl.num_programs(0) - 1)
    def _fin():
        den = jnp.maximum(accs[2], 1.0)
        lp_ref[...] = jnp.broadcast_to(
            accs[0] / den * (KD_T * KD_T) * LW_PIX, (1, 1))
        lr_ref[...] = jnp.broadcast_to(
            accs[1] / den * (KD_T * KD_T) * LW_REG, (1, 1))


def _loss(sa, ta, xpb, rp, mp, xrb, rr, mr, wa):
    f32 = jnp.float32
    blk = 128
    return pl.pallas_call(
        _loss_body,
        grid=(MAX_SAMPLES // blk,),
        in_specs=[
            pl.BlockSpec((blk, DIM), lambda i: (i, 0)),
            pl.BlockSpec((blk, DIM), lambda i: (i, 0)),
            pl.BlockSpec((NPIX_PAD, DIM), lambda i: (0, 0)),
            pl.BlockSpec((NPIX_PAD, DIM), lambda i: (0, 0)),
            pl.BlockSpec((NPIX_PAD, 1), lambda i: (0, 0)),
            pl.BlockSpec((NREG_PAD, DIM), lambda i: (0, 0)),
            pl.BlockSpec((NREG_PAD, DIM), lambda i: (0, 0)),
            pl.BlockSpec((NREG_PAD, 1), lambda i: (0, 0)),
            pl.BlockSpec((blk, 1), lambda i: (i, 0)),
        ],
        out_specs=[
            pl.BlockSpec((1, 1), lambda i: (0, 0)),
            pl.BlockSpec((1, 1), lambda i: (0, 0)),
        ],
        out_shape=[
            jax.ShapeDtypeStruct((1, 1), f32),
            jax.ShapeDtypeStruct((1, 1), f32),
        ],
        scratch_shapes=[
            pltpu.VMEM((NPIX_PAD, DIM), f32),
            pltpu.VMEM((NREG_PAD, DIM), f32),
            pltpu.SMEM((4,), f32),
        ],
    )(sa, ta, xpb, rp, mp, xrb, rr, mr, wa)


def kernel(s_feats, t_feats, logits_S, logits_T, labels, W1, gamma, beta, W2,
           seg_queue, pix_queue, seg_ptr, pix_ptr):
    f32 = jnp.float32
    n, _, h, w = s_feats.shape
    s_p_in = s_feats.transpose(0, 2, 3, 1).reshape(M, 512)
    t_p_in = t_feats.transpose(0, 2, 3, 1).reshape(M, DIM)
    lab = labels[:, 0, ::8, ::8].reshape(M)
    lab2d = lab.reshape(M, 1)

    sa, ta, mean_feat, upd, cnt_col = _prep(
        lab2d, s_p_in, t_p_in, W1, gamma.reshape(1, DIM),
        beta.reshape(1, DIM), W2)
    cnts = cnt_col[:, 0]

    pidx = jnp.asarray(_PIDX)
    ridx = jnp.asarray(_RIDX)
    cls_off = jnp.arange(NUM_CLASSES, dtype=jnp.int32)
    pflat = (cls_off[:, None] * PIXEL_MEM + pidx[None, :]).reshape(NPIX)
    rflat = (cls_off[:, None] * REGION_MEM + ridx[None, :]).reshape(NREG)
    pflat = jnp.concatenate(
        [pflat, jnp.zeros((NPIX_SC - NPIX,), jnp.int32)])
    rflat = jnp.concatenate(
        [rflat, jnp.zeros((NREG_SC - NREG,), jnp.int32)])
    xp_g, xr_g = _sc_gather(
        pix_queue.reshape(NUM_CLASSES * PIXEL_MEM, DIM), pflat,
        seg_queue.reshape(NUM_CLASSES * REGION_MEM, DIM), rflat)
    xpb = xp_g[:NPIX_PAD]
    xrb = xr_g[:NREG_PAD]

    kk = (pidx[None, :] - pix_ptr[:, None]) % PIXEL_MEM
    okp = (kk < PIX_UPD) & (kk.astype(f32) < cnts[:, None])
    src = cls_off[:, None] * PIX_UPD + jnp.minimum(kk, PIX_UPD - 1)
    rp = upd[src.reshape(NPIX)]
    mp = okp.reshape(NPIX, 1).astype(f32)
    rp = jnp.concatenate(
        [rp, jnp.zeros((NPIX_PAD - NPIX, DIM), f32)], axis=0)
    mp = jnp.concatenate(
        [mp, jnp.zeros((NPIX_PAD - NPIX, 1), f32)], axis=0)

    okr = (ridx[None, :] == seg_ptr[:, None]) & (cnts[:, None] > 0)
    rr = jnp.broadcast_to(mean_feat[:, None, :],
                          (NUM_CLASSES, REGION_CONTRAST, DIM)).reshape(NREG, DIM)
    mr = okr.reshape(NREG, 1).astype(f32)
    rr = jnp.concatenate(
        [rr, jnp.zeros((NREG_PAD - NREG, DIM), f32)], axis=0)
    mr = jnp.concatenate(
        [mr, jnp.zeros((NREG_PAD - NREG, 1), f32)], axis=0)

    wa = (lab[:MAX_SAMPLES] != IGNORE).astype(f32).reshape(MAX_SAMPLES, 1)

    lp_out, lr_out = _loss(sa, ta, xpb, rp, mp, xrb, rr, mr, wa)
    return (lp_out[0, 0], lr_out[0, 0])
